# baseline (speedup 1.0000x reference)
import functools

import jax
import jax.lax as lax
import jax.numpy as jnp
import numpy as np
from jax.experimental import pallas as pl
from jax.experimental.pallas import tpu as pltpu

_SLOTS = 1024
_DIM = 64
_GROUPS = 2
_TEMP = 0.4
_LOG_SLOTS = float(np.log(_SLOTS))
_TINY = float(np.finfo(np.float32).tiny)

_ROT_A = (13, 15, 26, 6)
_ROT_B = (17, 29, 16, 24)
_KS0 = np.uint32(0)
_KS1 = np.uint32(42)
_KS2 = np.uint32(0x1BD11BDA ^ 42)


@functools.lru_cache(maxsize=2)
def _gumbel_even_const(n_slabs: int, w: int):
    g = jax.random.gumbel(
        jax.random.key(42), (n_slabs * w, _SLOTS), dtype=jnp.float32
    )
    g = g.reshape(n_slabs, w, _SLOTS).transpose(0, 2, 1)
    return g[0::2]


def _tf_rounds(x0, x1, rots):
    for r in rots:
        x0 = x0 + x1
        x1 = (x1 << np.uint32(r)) | lax.shift_right_logical(
            x1, np.uint32(32 - r)
        )
        x1 = x1 ^ x0
    return x0, x1


def _gumbel_block(slab, w):
    k_iota = lax.broadcasted_iota(jnp.uint32, (_SLOTS, w), 0)
    t_iota = lax.broadcasted_iota(jnp.uint32, (_SLOTS, w), 1)
    base = (slab * (_SLOTS * w)).astype(jnp.uint32)
    lin = k_iota + t_iota * np.uint32(_SLOTS) + base
    x0 = jnp.zeros((_SLOTS, w), jnp.uint32)
    x1 = lin + _KS1
    x0, x1 = _tf_rounds(x0, x1, _ROT_A)
    x0 = x0 + _KS1
    x1 = x1 + (_KS2 + np.uint32(1))
    x0, x1 = _tf_rounds(x0, x1, _ROT_B)
    x0 = x0 + _KS2
    x1 = x1 + (_KS0 + np.uint32(2))
    x0, x1 = _tf_rounds(x0, x1, _ROT_A)
    x0 = x0 + _KS0
    x1 = x1 + (_KS1 + np.uint32(3))
    x0, x1 = _tf_rounds(x0, x1, _ROT_B)
    x0 = x0 + _KS1
    x1 = x1 + (_KS2 + np.uint32(4))
    x0, x1 = _tf_rounds(x0, x1, _ROT_A)
    x0 = x0 + _KS2
    x1 = x1 + (_KS0 + np.uint32(5))
    bits = x0 ^ x1
    fb = lax.shift_right_logical(bits, np.uint32(9)) | np.uint32(0x3F800000)
    f = lax.bitcast_convert_type(fb, jnp.float32) - 1.0
    u = jnp.maximum(f, jnp.float32(_TINY))
    return -jnp.log(-jnp.log(u))


def _copy(g_hbm, gbuf, sems, j, slot):
    return pltpu.make_async_copy(
        g_hbm.at[pl.ds(j, 1)], gbuf.at[pl.ds(slot, 1)], sems.at[slot]
    )


def _compute(i, z, cb, g, zq_ref, idx_ref, s_ref, m_ref):
    mm = jax.lax.dot_general(
        cb, z, (((1,), (0,)), ((), ())), preferred_element_type=jnp.float32
    )
    cb_sqr = jnp.sum(cb * cb, axis=1)[:, None]
    z_sqr = jnp.sum(z * z, axis=0)[None, :]
    logits = 2.0 * mm - cb_sqr - z_sqr

    y = (logits + g) * (1.0 / _TEMP)
    y_max = jnp.max(y, axis=0, keepdims=True)
    e = jnp.exp(y - y_max)
    s = jnp.sum(e, axis=0, keepdims=True)
    idx_ref[0, 0] = jnp.argmax(y, axis=0)

    zq_un = jax.lax.dot_general(
        cb, e, (((0,), (0,)), ((), ())), preferred_element_type=jnp.float32
    )
    zq_ref[0] = zq_un / s

    m2 = jnp.max(logits, axis=0, keepdims=True)
    e2 = jnp.exp(logits - m2)
    s2 = jnp.sum(e2, axis=0, keepdims=True)
    t = jnp.sum(e2 * logits, axis=0, keepdims=True)
    s_part = jnp.sum(t / s2, axis=1, keepdims=True)
    m_part = jnp.sum(m2 + jnp.log(s2), axis=1, keepdims=True)

    @pl.when(i == 0)
    def _init():
        s_ref[...] = jnp.zeros((1, 1), jnp.float32)
        m_ref[...] = jnp.zeros((1, 1), jnp.float32)

    s_ref[...] += s_part
    m_ref[...] += m_part


def _vq_block(z_ref, cb_ref, g_hbm, zq_ref, idx_ref, s_ref, m_ref, gbuf, sems):
    i = pl.program_id(0)
    z = z_ref[0]
    cb = cb_ref[...]
    w = z.shape[1]
    nj = pl.num_programs(0) // 2

    @pl.when(i == 0)
    def _first():
        _copy(g_hbm, gbuf, sems, 0, 0).start()

    @pl.when(lax.rem(i, 2) == 0)
    def _stream():
        j = lax.div(i, 2)
        slot = lax.rem(j, 2)
        _copy(g_hbm, gbuf, sems, j, slot).wait()

        @pl.when(j < nj - 1)
        def _prefetch():
            _copy(g_hbm, gbuf, sems, j + 1, lax.rem(j + 1, 2)).start()

        _compute(i, z, cb, gbuf[slot], zq_ref, idx_ref, s_ref, m_ref)

    @pl.when(lax.rem(i, 2) == 1)
    def _generate():
        _compute(i, z, cb, _gumbel_block(i, w), zq_ref, idx_ref, s_ref, m_ref)


def kernel(z_e, codebook):
    bs, feat_dim, w = z_e.shape
    n_slabs = bs * _GROUPS
    zr = z_e.reshape(n_slabs, _DIM, w)
    gumbel_even = _gumbel_even_const(n_slabs, w)

    zq, idx, s_tot, m_tot = pl.pallas_call(
        _vq_block,
        grid=(n_slabs,),
        in_specs=[
            pl.BlockSpec((1, _DIM, w), lambda i: (i, 0, 0)),
            pl.BlockSpec((_SLOTS, _DIM), lambda i: (0, 0)),
            pl.BlockSpec(memory_space=pltpu.MemorySpace.HBM),
        ],
        out_specs=[
            pl.BlockSpec((1, _DIM, w), lambda i: (i, 0, 0)),
            pl.BlockSpec((1, 1, w), lambda i: (i, 0, 0)),
            pl.BlockSpec((1, 1), lambda i: (0, 0)),
            pl.BlockSpec((1, 1), lambda i: (0, 0)),
        ],
        out_shape=[
            jax.ShapeDtypeStruct((n_slabs, _DIM, w), jnp.float32),
            jax.ShapeDtypeStruct((n_slabs, 1, w), jnp.int32),
            jax.ShapeDtypeStruct((1, 1), jnp.float32),
            jax.ShapeDtypeStruct((1, 1), jnp.float32),
        ],
        scratch_shapes=[
            pltpu.MemorySpace.VMEM((2, _SLOTS, w), jnp.float32),
            pltpu.SemaphoreType.DMA((2,)),
        ],
    )(zr, codebook, gumbel_even)

    n_rows = n_slabs * w
    denom = float(n_rows * _SLOTS)
    s0 = s_tot[0, 0]
    kl = (s0 - m_tot[0, 0] + n_rows * _LOG_SLOTS) / denom
    commit = -s0 / denom
    z_q = zq.reshape(bs, feat_dim, w)
    hard_indices = idx.reshape(bs, _GROUPS, w)
    return (z_q, hard_indices, kl, commit)


# speedup vs baseline: 1.0033x; 1.0033x over previous
import functools

import jax
import jax.lax as lax
import jax.numpy as jnp
import numpy as np
from jax.experimental import pallas as pl
from jax.experimental.pallas import tpu as pltpu

_SLOTS = 1024
_DIM = 64
_GROUPS = 2
_TEMP = 0.4
_LOG_SLOTS = float(np.log(_SLOTS))
_TINY = float(np.finfo(np.float32).tiny)

_ROT_A = (13, 15, 26, 6)
_ROT_B = (17, 29, 16, 24)
_KS0 = np.uint32(0)
_KS1 = np.uint32(42)
_KS2 = np.uint32(0x1BD11BDA ^ 42)


@functools.lru_cache(maxsize=2)
def _gumbel_even_const(n_slabs: int, w: int):
    g = jax.random.gumbel(
        jax.random.key(42), (n_slabs * w, _SLOTS), dtype=jnp.float32
    )
    g = g.reshape(n_slabs, w, _SLOTS).transpose(0, 2, 1)
    return g[0::2]


def _tf_rounds(x0, x1, rots):
    for r in rots:
        x0 = x0 + x1
        x1 = (x1 << np.uint32(r)) | lax.shift_right_logical(
            x1, np.uint32(32 - r)
        )
        x1 = x1 ^ x0
    return x0, x1


def _gumbel_block(slab, w):
    k_iota = lax.broadcasted_iota(jnp.uint32, (_SLOTS, w), 0)
    t_iota = lax.broadcasted_iota(jnp.uint32, (_SLOTS, w), 1)
    base = (slab * (_SLOTS * w)).astype(jnp.uint32)
    lin = k_iota + t_iota * np.uint32(_SLOTS) + base
    x0 = jnp.zeros((_SLOTS, w), jnp.uint32)
    x1 = lin + _KS1
    x0, x1 = _tf_rounds(x0, x1, _ROT_A)
    x0 = x0 + _KS1
    x1 = x1 + (_KS2 + np.uint32(1))
    x0, x1 = _tf_rounds(x0, x1, _ROT_B)
    x0 = x0 + _KS2
    x1 = x1 + (_KS0 + np.uint32(2))
    x0, x1 = _tf_rounds(x0, x1, _ROT_A)
    x0 = x0 + _KS0
    x1 = x1 + (_KS1 + np.uint32(3))
    x0, x1 = _tf_rounds(x0, x1, _ROT_B)
    x0 = x0 + _KS1
    x1 = x1 + (_KS2 + np.uint32(4))
    x0, x1 = _tf_rounds(x0, x1, _ROT_A)
    x0 = x0 + _KS2
    x1 = x1 + (_KS0 + np.uint32(5))
    bits = x0 ^ x1
    fb = lax.shift_right_logical(bits, np.uint32(9)) | np.uint32(0x3F800000)
    f = lax.bitcast_convert_type(fb, jnp.float32) - 1.0
    u = jnp.maximum(f, jnp.float32(_TINY))
    return -jnp.log(-jnp.log(u))


def _compute(p, i, z, cb, g, zq_ref, idx_ref, s_ref, m_ref):
    mm = jax.lax.dot_general(
        cb, z, (((1,), (0,)), ((), ())), preferred_element_type=jnp.float32
    )
    cb_sqr = jnp.sum(cb * cb, axis=1)[:, None]
    z_sqr = jnp.sum(z * z, axis=0)[None, :]
    logits = 2.0 * mm - cb_sqr - z_sqr

    y = (logits + g) * (1.0 / _TEMP)
    y_max = jnp.max(y, axis=0, keepdims=True)
    e = jnp.exp(y - y_max)
    s = jnp.sum(e, axis=0, keepdims=True)
    idx_ref[p, 0] = jnp.argmax(y, axis=0)

    zq_un = jax.lax.dot_general(
        cb, e, (((0,), (0,)), ((), ())), preferred_element_type=jnp.float32
    )
    zq_ref[p] = zq_un / s

    m2 = jnp.max(logits, axis=0, keepdims=True)
    e2 = jnp.exp(logits - m2)
    s2 = jnp.sum(e2, axis=0, keepdims=True)
    t = jnp.sum(e2 * logits, axis=0, keepdims=True)
    s_part = jnp.sum(t / s2, axis=1, keepdims=True)
    m_part = jnp.sum(m2 + jnp.log(s2), axis=1, keepdims=True)

    @pl.when(i == 0)
    def _init():
        s_ref[...] = jnp.zeros((1, 1), jnp.float32)
        m_ref[...] = jnp.zeros((1, 1), jnp.float32)

    s_ref[...] += s_part
    m_ref[...] += m_part


def _vq_block(z_ref, cb_ref, g_ref, zq_ref, idx_ref, s_ref, m_ref):
    j = pl.program_id(0)
    cb = cb_ref[...]
    w = z_ref.shape[2]
    _compute(0, 2 * j, z_ref[0], cb, g_ref[0], zq_ref, idx_ref, s_ref, m_ref)
    _compute(
        1, 2 * j + 1, z_ref[1], cb, _gumbel_block(2 * j + 1, w),
        zq_ref, idx_ref, s_ref, m_ref,
    )


def kernel(z_e, codebook):
    bs, feat_dim, w = z_e.shape
    n_slabs = bs * _GROUPS
    zr = z_e.reshape(n_slabs, _DIM, w)
    gumbel_even = _gumbel_even_const(n_slabs, w)

    zq, idx, s_tot, m_tot = pl.pallas_call(
        _vq_block,
        grid=(n_slabs // 2,),
        in_specs=[
            pl.BlockSpec((2, _DIM, w), lambda j: (j, 0, 0)),
            pl.BlockSpec((_SLOTS, _DIM), lambda j: (0, 0)),
            pl.BlockSpec((1, _SLOTS, w), lambda j: (j, 0, 0)),
        ],
        out_specs=[
            pl.BlockSpec((2, _DIM, w), lambda j: (j, 0, 0)),
            pl.BlockSpec((2, 1, w), lambda j: (j, 0, 0)),
            pl.BlockSpec((1, 1), lambda j: (0, 0)),
            pl.BlockSpec((1, 1), lambda j: (0, 0)),
        ],
        out_shape=[
            jax.ShapeDtypeStruct((n_slabs, _DIM, w), jnp.float32),
            jax.ShapeDtypeStruct((n_slabs, 1, w), jnp.int32),
            jax.ShapeDtypeStruct((1, 1), jnp.float32),
            jax.ShapeDtypeStruct((1, 1), jnp.float32),
        ],
    )(zr, codebook, gumbel_even)

    n_rows = n_slabs * w
    denom = float(n_rows * _SLOTS)
    s0 = s_tot[0, 0]
    kl = (s0 - m_tot[0, 0] + n_rows * _LOG_SLOTS) / denom
    commit = -s0 / denom
    z_q = zq.reshape(bs, feat_dim, w)
    hard_indices = idx.reshape(bs, _GROUPS, w)
    return (z_q, hard_indices, kl, commit)
